# Initial kernel scaffold; baseline (speedup 1.0000x reference)
import jax
import jax.numpy as jnp
from jax.experimental import pallas as pl


def kernel(u_feat, f_feat, edge_index, edge_w, edge_u, edge_f, Wu, bu, Wf, bf, W1, b1, W2, b2, g1, be1, g2, be2, Wr, br):
    raise NotImplementedError("write your pallas kernel here")


# speedup vs baseline: 1.1189x; 1.1189x over previous
import functools

import jax
import jax.numpy as jnp
from jax import lax
from jax.experimental import pallas as pl
from jax.experimental.pallas import tpu as pltpu
from jax.experimental.pallas import tpu_sc as plsc

N_U = 5000
N = 10000
E = 160000
HID = 512
FC = 128
NCHUNK = HID // FC
N_PAD = 10240
E_PAD = 163840
E_PAD2 = 160256
LRELU = 0.01
LN_EPS = 1e-5
MAX_RATING = 5.0

_f32 = jnp.float32
_i32 = jnp.int32


def _mm_bias_body(x_ref, w_ref, b_ref, o_ref):
    o_ref[...] = jnp.dot(x_ref[...], w_ref[...],
                         preferred_element_type=_f32) + b_ref[...]


def _mm_bias(x, w, b, blk=256):
    m = x.shape[0]
    return pl.pallas_call(
        _mm_bias_body,
        grid=(m // blk,),
        in_specs=[
            pl.BlockSpec((blk, x.shape[1]), lambda i: (i, 0)),
            pl.BlockSpec((w.shape[0], w.shape[1]), lambda i: (0, 0)),
            pl.BlockSpec((1, w.shape[1]), lambda i: (0, 0)),
        ],
        out_specs=pl.BlockSpec((blk, w.shape[1]), lambda i: (i, 0)),
        out_shape=jax.ShapeDtypeStruct((m, w.shape[1]), _f32),
    )(x, w, b.reshape(1, -1))


def _chunk_mm_body(x_ref, w_ref, o_ref):
    o_ref[0] = jnp.dot(x_ref[...], w_ref[...], preferred_element_type=_f32)


def _chunk_mm(x, w, blk=256):
    return pl.pallas_call(
        _chunk_mm_body,
        grid=(N_PAD // blk, NCHUNK),
        in_specs=[
            pl.BlockSpec((blk, HID), lambda m, c: (m, 0)),
            pl.BlockSpec((HID, FC), lambda m, c: (0, c)),
        ],
        out_specs=pl.BlockSpec((1, blk, FC), lambda m, c: (c, m, 0)),
        out_shape=jax.ShapeDtypeStruct((NCHUNK, N_PAD, FC), _f32),
    )(x, w)


def _gcn_epilogue(s_ref, h_ref, d_ref, b_ref, g_ref, be_ref):
    s = jnp.concatenate([s_ref[i] for i in range(NCHUNK)], axis=-1)
    h = jnp.concatenate([h_ref[i] for i in range(NCHUNK)], axis=-1)
    di = d_ref[0, 0]
    gcn = s + (di * di)[:, None] * h + b_ref[...]
    mu = jnp.mean(gcn, axis=-1, keepdims=True)
    var = jnp.mean((gcn - mu) ** 2, axis=-1, keepdims=True)
    xn = (gcn - mu) * lax.rsqrt(var + LN_EPS) * g_ref[...] + be_ref[...]
    return jnp.where(xn >= 0, xn, LRELU * xn)


def _post1_body(s_ref, h_ref, d_ref, b_ref, g_ref, be_ref, w_ref, o_ref):
    x = _gcn_epilogue(s_ref, h_ref, d_ref, b_ref, g_ref, be_ref)
    y = jnp.dot(x, w_ref[...], preferred_element_type=_f32)
    for c in range(NCHUNK):
        o_ref[c] = y[:, c * FC:(c + 1) * FC]


def _post1(scat, h, dinv, b, g, be, w_next, blk=256):
    return pl.pallas_call(
        _post1_body,
        grid=(N_PAD // blk,),
        in_specs=[
            pl.BlockSpec((NCHUNK, blk, FC), lambda m: (0, m, 0)),
            pl.BlockSpec((NCHUNK, blk, FC), lambda m: (0, m, 0)),
            pl.BlockSpec((1, 1, blk), lambda m: (m, 0, 0)),
            pl.BlockSpec((1, HID), lambda m: (0, 0)),
            pl.BlockSpec((1, HID), lambda m: (0, 0)),
            pl.BlockSpec((1, HID), lambda m: (0, 0)),
            pl.BlockSpec((HID, HID), lambda m: (0, 0)),
        ],
        out_specs=pl.BlockSpec((NCHUNK, blk, FC), lambda m: (0, m, 0)),
        out_shape=jax.ShapeDtypeStruct((NCHUNK, N_PAD, FC), _f32),
    )(scat, h, dinv.reshape(N_PAD // blk, 1, blk), b.reshape(1, -1),
      g.reshape(1, -1), be.reshape(1, -1), w_next)


def _post2_body(s_ref, h_ref, d_ref, b_ref, g_ref, be_ref, w_ref, br_ref,
                su_ref, sf_ref):
    x = _gcn_epilogue(s_ref, h_ref, d_ref, b_ref, g_ref, be_ref)
    y = jnp.dot(x, w_ref[...], preferred_element_type=_f32)
    su_ref[...] = y[:, 0:1] + br_ref[...]
    sf_ref[...] = y[:, 1:2]


def _post2(scat, h, dinv, b, g, be, wr2, br, blk=256):
    return pl.pallas_call(
        _post2_body,
        grid=(N_PAD // blk,),
        in_specs=[
            pl.BlockSpec((NCHUNK, blk, FC), lambda m: (0, m, 0)),
            pl.BlockSpec((NCHUNK, blk, FC), lambda m: (0, m, 0)),
            pl.BlockSpec((1, 1, blk), lambda m: (m, 0, 0)),
            pl.BlockSpec((1, HID), lambda m: (0, 0)),
            pl.BlockSpec((1, HID), lambda m: (0, 0)),
            pl.BlockSpec((1, HID), lambda m: (0, 0)),
            pl.BlockSpec((HID, 2), lambda m: (0, 0)),
            pl.BlockSpec((1, 1), lambda m: (0, 0)),
        ],
        out_specs=[
            pl.BlockSpec((blk, 1), lambda m: (m, 0)),
            pl.BlockSpec((blk, 1), lambda m: (m, 0)),
        ],
        out_shape=[
            jax.ShapeDtypeStruct((N_PAD, 1), _f32),
            jax.ShapeDtypeStruct((N_PAD, 1), _f32),
        ],
    )(scat, h, dinv.reshape(N_PAD // blk, 1, blk), b.reshape(1, -1),
      g.reshape(1, -1), be.reshape(1, -1), wr2, br.reshape(1, 1))


_MESH = plsc.VectorSubcoreMesh(core_axis_name="c", subcore_axis_name="s",
                               num_cores=2, num_subcores=16)
_NC, _NS = 2, 16
_NW = _NC * _NS

_EB32 = E_PAD // _NW // 128
_EB16 = E_PAD // _NS // 128
_ROWS_T = N_PAD // _NS


def _full16(v, dtype=_i32):
    return jnp.full((16,), v, dtype=dtype)


def _newton_rsqrt(x):
    i = plsc.bitcast(x, _i32)
    i = jnp.full((16,), 0x5F3759DF, _i32) - (i >> 1)
    y = plsc.bitcast(i, _f32)
    for _ in range(3):
        y = y * (1.5 - 0.5 * x * y * y)
    return y


def _sc_deg_body(col3_hbm, ew_hbm, deg_hbm, col_v, ew_v, val_v, zv_v, acc_sh):
    c = lax.axis_index("c")
    s = lax.axis_index("s")
    wid = s * _NC + c

    def z16(m, _):
        zv_v[pl.ds(16 * m, 16)] = jnp.zeros((16,), _f32)
        return _
    lax.fori_loop(0, 8, z16, None)
    for p in range(_ROWS_T // 128):
        pltpu.sync_copy(zv_v, acc_sh.at[pl.ds(s * _ROWS_T + p * 128, 128)])
    plsc.subcore_barrier()

    pltpu.sync_copy(col3_hbm.at[pl.ds(wid * _EB32, _EB32)], col_v)
    pltpu.sync_copy(ew_hbm.at[pl.ds(wid * _EB32 * 128, _EB32 * 128)], ew_v)

    def blk(j, _):
        def cpv(m, _):
            val_v[pl.ds(16 * m, 16)] = ew_v[pl.ds(j * 128 + 16 * m, 16)]
            return _
        lax.fori_loop(0, 8, cpv, None)
        pltpu.sync_copy(val_v, acc_sh.at[col_v.at[j]], add=True)
        return _
    lax.fori_loop(0, _EB32, blk, None)
    plsc.subcore_barrier()

    for p in range(_ROWS_T // 128):
        pltpu.sync_copy(acc_sh.at[pl.ds(s * _ROWS_T + p * 128, 128)], zv_v)
        pltpu.sync_copy(
            zv_v,
            deg_hbm.at[pl.ds(c * N_PAD + s * _ROWS_T + p * 128, 128)])


def _sc_deg(col2, ew2):
    f = pl.kernel(
        _sc_deg_body,
        out_type=jax.ShapeDtypeStruct((2 * N_PAD,), _f32),
        mesh=_MESH,
        compiler_params=pltpu.CompilerParams(needs_layout_passes=False),
        scratch_types=[
            pltpu.VMEM((_EB32, 128), _i32),
            pltpu.VMEM((_EB32 * 128,), _f32),
            pltpu.VMEM((128,), _f32),
            pltpu.VMEM((128,), _f32),
            pltpu.VMEM_SHARED((N_PAD,), _f32),
        ],
    )
    return f(col2, ew2.reshape(-1))


def _sc_norm_body(deg_hbm, row_hbm, col_hbm, ew_hbm, dinv_hbm, norm_hbm,
                  v0_v, v1_v, dmine_v, dfull_v, row_v, col_v, ew_v, norm_v):
    c = lax.axis_index("c")
    s = lax.axis_index("s")
    wid = s * _NC + c

    pltpu.sync_copy(deg_hbm.at[pl.ds(s * _ROWS_T, _ROWS_T)], v0_v)
    pltpu.sync_copy(deg_hbm.at[pl.ds(N_PAD + s * _ROWS_T, _ROWS_T)], v1_v)

    def dv(v, _):
        d = v0_v[pl.ds(v * 16, 16)] + v1_v[pl.ds(v * 16, 16)] + 1.0
        dmine_v[pl.ds(v * 16, 16)] = _newton_rsqrt(d)
        return _
    lax.fori_loop(0, _ROWS_T // 16, dv, None)

    pltpu.sync_copy(dmine_v,
                    dinv_hbm.at[pl.ds(c * N_PAD + s * _ROWS_T, _ROWS_T)])
    plsc.subcore_barrier()
    pltpu.sync_copy(dinv_hbm.at[pl.ds(c * N_PAD, N_PAD)], dfull_v)

    pltpu.sync_copy(row_hbm.at[pl.ds(wid * _EB32, _EB32)], row_v)
    pltpu.sync_copy(col_hbm.at[pl.ds(wid * _EB32, _EB32)], col_v)
    pltpu.sync_copy(ew_hbm.at[pl.ds(wid * _EB32, _EB32)], ew_v)

    def blk(j, _):
        for m in range(8):
            r16 = row_v[j, pl.ds(16 * m, 16)]
            c16 = col_v[j, pl.ds(16 * m, 16)]
            w16 = ew_v[j, pl.ds(16 * m, 16)]
            dr = plsc.load_gather(dfull_v, [r16])
            dc = plsc.load_gather(dfull_v, [c16])
            norm_v[j, pl.ds(16 * m, 16)] = dr * w16 * dc
        return _
    lax.fori_loop(0, _EB32, blk, None)
    pltpu.sync_copy(norm_v, norm_hbm.at[pl.ds(wid * _EB32, _EB32)])


def _sc_norm(deg_parts, row2, col2, ew2):
    f = pl.kernel(
        _sc_norm_body,
        out_type=[
            jax.ShapeDtypeStruct((2 * N_PAD,), _f32),
            jax.ShapeDtypeStruct((E_PAD // 128, 128), _f32),
        ],
        mesh=_MESH,
        compiler_params=pltpu.CompilerParams(needs_layout_passes=False),
        scratch_types=[
            pltpu.VMEM((_ROWS_T,), _f32),
            pltpu.VMEM((_ROWS_T,), _f32),
            pltpu.VMEM((_ROWS_T,), _f32),
            pltpu.VMEM((N_PAD,), _f32),
            pltpu.VMEM((_EB32, 128), _i32),
            pltpu.VMEM((_EB32, 128), _i32),
            pltpu.VMEM((_EB32, 128), _f32),
            pltpu.VMEM((_EB32, 128), _f32),
        ],
    )
    return f(deg_parts, row2, col2, ew2)


def _sc_msg_body(h_hbm, norm_hbm, row_hbm, col3_hbm, scat_hbm,
                 row_v, col_v, norm_v, rows_v, sem, acc_sh):
    c = lax.axis_index("c")
    s = lax.axis_index("s")

    pltpu.sync_copy(row_hbm.at[pl.ds(s * _EB16, _EB16)], row_v)
    pltpu.sync_copy(col3_hbm.at[pl.ds(s * _EB16, _EB16)], col_v)
    pltpu.sync_copy(norm_hbm.at[pl.ds(s * _EB16 * 128, _EB16 * 128)], norm_v)

    z16 = jnp.zeros((16,), _f32)

    for k in range(NCHUNK // _NC):
        chunk = c * (NCHUNK // _NC) + k
        off = jnp.where(k == 0, c * (NCHUNK // _NC) * N_PAD, N_PAD).astype(_i32)

        def mkidx(j, _):
            for m in range(8):
                row_v[j, pl.ds(16 * m, 16)] = row_v[j, pl.ds(16 * m, 16)] + off
            return _
        lax.fori_loop(0, _EB16, mkidx, None)

        def zrow(j, _):
            for m in range(FC // 16):
                rows_v[j, pl.ds(16 * m, 16)] = z16
            return _
        lax.fori_loop(0, 128, zrow, None)
        for p in range(_ROWS_T // 128):
            pltpu.sync_copy(rows_v, acc_sh.at[pl.ds(s * _ROWS_T + p * 128, 128)])
        plsc.subcore_barrier()

        def blk(j, _):
            pltpu.async_copy(h_hbm.at[row_v.at[j]], rows_v, sem).wait()

            def scale(e, _):
                nb = plsc.load_gather(norm_v, [_full16(j * 128 + e)])
                for f in range(FC // 16):
                    rows_v[e, pl.ds(16 * f, 16)] = (
                        rows_v[e, pl.ds(16 * f, 16)] * nb)
                return _
            lax.fori_loop(0, 128, scale, None)
            pltpu.sync_copy(rows_v, acc_sh.at[col_v.at[j]], add=True)
            return _
        lax.fori_loop(0, _EB16, blk, None)
        plsc.subcore_barrier()

        coff = chunk * N_PAD
        for p in range(_ROWS_T // 128):
            pltpu.sync_copy(acc_sh.at[pl.ds(s * _ROWS_T + p * 128, 128)], rows_v)
            pltpu.sync_copy(
                rows_v,
                scat_hbm.at[pl.ds(coff + s * _ROWS_T + p * 128, 128)])


def _sc_msg(h_flat, norm2, row2, col2):
    f = pl.kernel(
        _sc_msg_body,
        out_type=jax.ShapeDtypeStruct((NCHUNK * N_PAD, FC), _f32),
        mesh=_MESH,
        compiler_params=pltpu.CompilerParams(needs_layout_passes=False),
        scratch_types=[
            pltpu.VMEM((_EB16, 128), _i32),
            pltpu.VMEM((_EB16, 1, 128), _i32),
            pltpu.VMEM((_EB16 * 128,), _f32),
            pltpu.VMEM((128, FC), _f32),
            pltpu.SemaphoreType.DMA,
            pltpu.VMEM_SHARED((N_PAD, FC), _f32),
        ],
    )
    return f(h_flat, norm2.reshape(-1), row2,
             col2.reshape(_EB16 * _NS, 1, 128))


_ET32 = E_PAD2 // _NW


def _sc_score_body(su_hbm, sf_hbm, eu_hbm, ef_hbm, out_hbm,
                   su_v, sf_v, eu_v, ef_v, out_v):
    c = lax.axis_index("c")
    s = lax.axis_index("s")
    wid = s * _NC + c
    base = wid * _ET32

    pltpu.sync_copy(su_hbm, su_v)
    pltpu.sync_copy(sf_hbm, sf_v)
    pltpu.sync_copy(eu_hbm.at[pl.ds(base, _ET32)], eu_v)
    pltpu.sync_copy(ef_hbm.at[pl.ds(base, _ET32)], ef_v)

    def vec(i, _):
        eu16 = eu_v[pl.ds(16 * i, 16)]
        ef16 = ef_v[pl.ds(16 * i, 16)]
        a = plsc.load_gather(su_v, [eu16])
        b = plsc.load_gather(sf_v, [ef16])
        out_v[pl.ds(16 * i, 16)] = MAX_RATING / (1.0 + jnp.exp(-(a + b)))
        return _
    lax.fori_loop(0, _ET32 // 16, vec, None)
    pltpu.sync_copy(out_v, out_hbm.at[pl.ds(base, _ET32)])


def _sc_score(su, sf, eu_p, ef_p):
    f = pl.kernel(
        _sc_score_body,
        out_type=jax.ShapeDtypeStruct((E_PAD2,), _f32),
        mesh=_MESH,
        compiler_params=pltpu.CompilerParams(needs_layout_passes=False),
        scratch_types=[
            pltpu.VMEM((N_PAD,), _f32),
            pltpu.VMEM((N_PAD,), _f32),
            pltpu.VMEM((_ET32,), _i32),
            pltpu.VMEM((_ET32,), _i32),
            pltpu.VMEM((_ET32,), _f32),
        ],
    )
    return f(su, sf, eu_p, ef_p)


def kernel(u_feat, f_feat, edge_index, edge_w, edge_u, edge_f,
           Wu, bu, Wf, bf, W1, b1, W2, b2, g1, be1, g2, be2, Wr, br):
    row = edge_index[0]
    col = edge_index[1]

    ep = E_PAD - E
    row_p = jnp.concatenate([row, jnp.zeros((ep,), _i32)])
    col_p = jnp.concatenate([col, jnp.full((ep,), 10200, _i32)])
    ew_p = jnp.concatenate([edge_w, jnp.zeros((ep,), _f32)])
    row2 = row_p.reshape(E_PAD // 128, 128)
    col2 = col_p.reshape(E_PAD // 128, 128)
    ew2 = ew_p.reshape(E_PAD // 128, 128)

    ep2 = E_PAD2 - E
    eu_p = jnp.concatenate([edge_u, jnp.zeros((ep2,), _i32)])
    ef_p = jnp.concatenate([edge_f, jnp.full((ep2,), N_U, _i32)])

    u_pad = jnp.concatenate([u_feat, jnp.zeros((120, u_feat.shape[1]), _f32)])
    f_pad = jnp.concatenate([f_feat, jnp.zeros((120, f_feat.shape[1]), _f32)])
    xu = _mm_bias(u_pad, Wu, bu)
    xf = _mm_bias(f_pad, Wf, bf)
    x0 = jnp.concatenate([xu[:N_U], xf[:N_U],
                          jnp.zeros((N_PAD - N, HID), _f32)], axis=0)
    h1 = _chunk_mm(x0, W1)

    deg_parts = _sc_deg(col2, ew2)
    dinv2, norm2 = _sc_norm(deg_parts, row2, col2, ew2)
    dinv = dinv2[:N_PAD]

    normf = norm2.reshape(-1)

    def _msg(h4):
        hh = h4.reshape(NCHUNK, N_PAD, FC)
        out = []
        for cc in range(NCHUNK):
            msg = hh[cc][row_p] * normf[:, None]
            out.append(jnp.zeros((N_PAD, FC), _f32).at[col_p].add(msg))
        return jnp.stack(out)

    scat1 = _msg(h1)
    h2 = _post1(scat1.reshape(NCHUNK, N_PAD, FC), h1, dinv, b1, g1, be1, W2)

    scat2 = _msg(h2)
    wr2 = Wr.reshape(2, HID, 1)
    wr2 = jnp.concatenate([wr2[0], wr2[1]], axis=1)
    su, sf = _post2(scat2.reshape(NCHUNK, N_PAD, FC), h2, dinv,
                    b2, g2, be2, wr2, br)

    score_p = _sc_score(su.reshape(-1), sf.reshape(-1), eu_p, ef_p)
    return score_p[:E]


# speedup vs baseline: 1.1190x; 1.0001x over previous
import jax
import jax.numpy as jnp
from jax import lax
from jax.experimental import pallas as pl
from jax.experimental.pallas import tpu as pltpu
from jax.experimental.pallas import tpu_sc as plsc

N_U = 5000
N = 10000
E = 160000
HID = 512
FC = 128
NCHUNK = HID // FC
N_PAD = 10240
E_PAD = 163840
E_PAD2 = 160256
LRELU = 0.01
LN_EPS = 1e-5
MAX_RATING = 5.0

_f32 = jnp.float32
_i32 = jnp.int32


def _mm_bias_body(x_ref, w_ref, b_ref, o_ref):
    o_ref[...] = jnp.dot(x_ref[...], w_ref[...],
                         preferred_element_type=_f32) + b_ref[...]


def _mm_bias(x, w, b, blk=256):
    m = x.shape[0]
    return pl.pallas_call(
        _mm_bias_body,
        grid=(m // blk,),
        in_specs=[
            pl.BlockSpec((blk, x.shape[1]), lambda i: (i, 0)),
            pl.BlockSpec((w.shape[0], w.shape[1]), lambda i: (0, 0)),
            pl.BlockSpec((1, w.shape[1]), lambda i: (0, 0)),
        ],
        out_specs=pl.BlockSpec((blk, w.shape[1]), lambda i: (i, 0)),
        out_shape=jax.ShapeDtypeStruct((m, w.shape[1]), _f32),
    )(x, w, b.reshape(1, -1))


def _chunk_mm_body(x_ref, w_ref, o_ref):
    o_ref[0] = jnp.dot(x_ref[...], w_ref[...], preferred_element_type=_f32)


def _chunk_mm(x, w, blk=256):
    return pl.pallas_call(
        _chunk_mm_body,
        grid=(N_PAD // blk, NCHUNK),
        in_specs=[
            pl.BlockSpec((blk, HID), lambda m, c: (m, 0)),
            pl.BlockSpec((HID, FC), lambda m, c: (0, c)),
        ],
        out_specs=pl.BlockSpec((1, blk, FC), lambda m, c: (c, m, 0)),
        out_shape=jax.ShapeDtypeStruct((NCHUNK, N_PAD, FC), _f32),
    )(x, w)


def _gcn_epilogue(s_ref, h_ref, d_ref, b_ref, g_ref, be_ref):
    s = jnp.concatenate([s_ref[i] for i in range(NCHUNK)], axis=-1)
    h = jnp.concatenate([h_ref[i] for i in range(NCHUNK)], axis=-1)
    di = d_ref[0, 0]
    gcn = s + (di * di)[:, None] * h + b_ref[...]
    mu = jnp.mean(gcn, axis=-1, keepdims=True)
    var = jnp.mean((gcn - mu) ** 2, axis=-1, keepdims=True)
    xn = (gcn - mu) * lax.rsqrt(var + LN_EPS) * g_ref[...] + be_ref[...]
    return jnp.where(xn >= 0, xn, LRELU * xn)


def _post1_body(s_ref, h_ref, d_ref, b_ref, g_ref, be_ref, w_ref, o_ref):
    x = _gcn_epilogue(s_ref, h_ref, d_ref, b_ref, g_ref, be_ref)
    y = jnp.dot(x, w_ref[...], preferred_element_type=_f32)
    for c in range(NCHUNK):
        o_ref[c] = y[:, c * FC:(c + 1) * FC]


def _post1(scat, h, dinv, b, g, be, w_next, blk=256):
    return pl.pallas_call(
        _post1_body,
        grid=(N_PAD // blk,),
        in_specs=[
            pl.BlockSpec((NCHUNK, blk, FC), lambda m: (0, m, 0)),
            pl.BlockSpec((NCHUNK, blk, FC), lambda m: (0, m, 0)),
            pl.BlockSpec((1, 1, blk), lambda m: (m, 0, 0)),
            pl.BlockSpec((1, HID), lambda m: (0, 0)),
            pl.BlockSpec((1, HID), lambda m: (0, 0)),
            pl.BlockSpec((1, HID), lambda m: (0, 0)),
            pl.BlockSpec((HID, HID), lambda m: (0, 0)),
        ],
        out_specs=pl.BlockSpec((NCHUNK, blk, FC), lambda m: (0, m, 0)),
        out_shape=jax.ShapeDtypeStruct((NCHUNK, N_PAD, FC), _f32),
    )(scat, h, dinv.reshape(N_PAD // blk, 1, blk), b.reshape(1, -1),
      g.reshape(1, -1), be.reshape(1, -1), w_next)


def _post2_body(s_ref, h_ref, d_ref, b_ref, g_ref, be_ref, w_ref, br_ref,
                su_ref, sf_ref):
    x = _gcn_epilogue(s_ref, h_ref, d_ref, b_ref, g_ref, be_ref)
    y = jnp.dot(x, w_ref[...], preferred_element_type=_f32)
    su_ref[...] = y[:, 0:1] + br_ref[...]
    sf_ref[...] = y[:, 1:2]


def _post2(scat, h, dinv, b, g, be, wr2, br, blk=256):
    return pl.pallas_call(
        _post2_body,
        grid=(N_PAD // blk,),
        in_specs=[
            pl.BlockSpec((NCHUNK, blk, FC), lambda m: (0, m, 0)),
            pl.BlockSpec((NCHUNK, blk, FC), lambda m: (0, m, 0)),
            pl.BlockSpec((1, 1, blk), lambda m: (m, 0, 0)),
            pl.BlockSpec((1, HID), lambda m: (0, 0)),
            pl.BlockSpec((1, HID), lambda m: (0, 0)),
            pl.BlockSpec((1, HID), lambda m: (0, 0)),
            pl.BlockSpec((HID, 2), lambda m: (0, 0)),
            pl.BlockSpec((1, 1), lambda m: (0, 0)),
        ],
        out_specs=[
            pl.BlockSpec((blk, 1), lambda m: (m, 0)),
            pl.BlockSpec((blk, 1), lambda m: (m, 0)),
        ],
        out_shape=[
            jax.ShapeDtypeStruct((N_PAD, 1), _f32),
            jax.ShapeDtypeStruct((N_PAD, 1), _f32),
        ],
    )(scat, h, dinv.reshape(N_PAD // blk, 1, blk), b.reshape(1, -1),
      g.reshape(1, -1), be.reshape(1, -1), wr2, br.reshape(1, 1))


_MESH = plsc.VectorSubcoreMesh(core_axis_name="c", subcore_axis_name="s",
                               num_cores=2, num_subcores=16)
_NC, _NS = 2, 16
_NW = _NC * _NS

_EB32 = E_PAD // _NW // 128
_ROWS_T = N_PAD // _NS


def _newton_rsqrt(x):
    i = plsc.bitcast(x, _i32)
    i = jnp.full((16,), 0x5F3759DF, _i32) - (i >> 1)
    y = plsc.bitcast(i, _f32)
    for _ in range(3):
        y = y * (1.5 - 0.5 * x * y * y)
    return y


def _sc_deg_body(col3_hbm, ew_hbm, deg_hbm, col_v, ew_v, val_v, zv_v, acc_sh):
    c = lax.axis_index("c")
    s = lax.axis_index("s")
    wid = s * _NC + c

    def z16(m, _):
        zv_v[pl.ds(16 * m, 16)] = jnp.zeros((16,), _f32)
        return _
    lax.fori_loop(0, 8, z16, None)
    for p in range(_ROWS_T // 128):
        pltpu.sync_copy(zv_v, acc_sh.at[pl.ds(s * _ROWS_T + p * 128, 128)])
    plsc.subcore_barrier()

    pltpu.sync_copy(col3_hbm.at[pl.ds(wid * _EB32, _EB32)], col_v)
    pltpu.sync_copy(ew_hbm.at[pl.ds(wid * _EB32 * 128, _EB32 * 128)], ew_v)

    def blk(j, _):
        def cpv(m, _):
            val_v[pl.ds(16 * m, 16)] = ew_v[pl.ds(j * 128 + 16 * m, 16)]
            return _
        lax.fori_loop(0, 8, cpv, None)
        pltpu.sync_copy(val_v, acc_sh.at[col_v.at[j]], add=True)
        return _
    lax.fori_loop(0, _EB32, blk, None)
    plsc.subcore_barrier()

    for p in range(_ROWS_T // 128):
        pltpu.sync_copy(acc_sh.at[pl.ds(s * _ROWS_T + p * 128, 128)], zv_v)
        pltpu.sync_copy(
            zv_v,
            deg_hbm.at[pl.ds(c * N_PAD + s * _ROWS_T + p * 128, 128)])


def _sc_deg(col2, ew2):
    f = pl.kernel(
        _sc_deg_body,
        out_type=jax.ShapeDtypeStruct((2 * N_PAD,), _f32),
        mesh=_MESH,
        compiler_params=pltpu.CompilerParams(needs_layout_passes=False),
        scratch_types=[
            pltpu.VMEM((_EB32, 128), _i32),
            pltpu.VMEM((_EB32 * 128,), _f32),
            pltpu.VMEM((128,), _f32),
            pltpu.VMEM((128,), _f32),
            pltpu.VMEM_SHARED((N_PAD,), _f32),
        ],
    )
    return f(col2, ew2.reshape(-1))


def _sc_norm_body(deg_hbm, row_hbm, col_hbm, ew_hbm, dinv_hbm, norm_hbm,
                  v0_v, v1_v, dmine_v, dfull_v, row_v, col_v, ew_v, norm_v):
    c = lax.axis_index("c")
    s = lax.axis_index("s")
    wid = s * _NC + c

    pltpu.sync_copy(deg_hbm.at[pl.ds(s * _ROWS_T, _ROWS_T)], v0_v)
    pltpu.sync_copy(deg_hbm.at[pl.ds(N_PAD + s * _ROWS_T, _ROWS_T)], v1_v)

    def dv(v, _):
        d = v0_v[pl.ds(v * 16, 16)] + v1_v[pl.ds(v * 16, 16)] + 1.0
        dmine_v[pl.ds(v * 16, 16)] = _newton_rsqrt(d)
        return _
    lax.fori_loop(0, _ROWS_T // 16, dv, None)

    pltpu.sync_copy(dmine_v,
                    dinv_hbm.at[pl.ds(c * N_PAD + s * _ROWS_T, _ROWS_T)])
    plsc.subcore_barrier()
    pltpu.sync_copy(dinv_hbm.at[pl.ds(c * N_PAD, N_PAD)], dfull_v)

    pltpu.sync_copy(row_hbm.at[pl.ds(wid * _EB32, _EB32)], row_v)
    pltpu.sync_copy(col_hbm.at[pl.ds(wid * _EB32, _EB32)], col_v)
    pltpu.sync_copy(ew_hbm.at[pl.ds(wid * _EB32, _EB32)], ew_v)

    def blk(j, _):
        for m in range(8):
            r16 = row_v[j, pl.ds(16 * m, 16)]
            c16 = col_v[j, pl.ds(16 * m, 16)]
            w16 = ew_v[j, pl.ds(16 * m, 16)]
            dr = plsc.load_gather(dfull_v, [r16])
            dc = plsc.load_gather(dfull_v, [c16])
            norm_v[j, pl.ds(16 * m, 16)] = dr * w16 * dc
        return _
    lax.fori_loop(0, _EB32, blk, None)
    pltpu.sync_copy(norm_v, norm_hbm.at[pl.ds(wid * _EB32, _EB32)])


def _sc_norm(deg_parts, row2, col2, ew2):
    f = pl.kernel(
        _sc_norm_body,
        out_type=[
            jax.ShapeDtypeStruct((2 * N_PAD,), _f32),
            jax.ShapeDtypeStruct((E_PAD // 128, 128), _f32),
        ],
        mesh=_MESH,
        compiler_params=pltpu.CompilerParams(needs_layout_passes=False),
        scratch_types=[
            pltpu.VMEM((_ROWS_T,), _f32),
            pltpu.VMEM((_ROWS_T,), _f32),
            pltpu.VMEM((_ROWS_T,), _f32),
            pltpu.VMEM((N_PAD,), _f32),
            pltpu.VMEM((_EB32, 128), _i32),
            pltpu.VMEM((_EB32, 128), _i32),
            pltpu.VMEM((_EB32, 128), _f32),
            pltpu.VMEM((_EB32, 128), _f32),
        ],
    )
    return f(deg_parts, row2, col2, ew2)


_ET32 = E_PAD2 // _NW


def _sc_score_body(su_hbm, sf_hbm, eu_hbm, ef_hbm, out_hbm,
                   su_v, sf_v, eu_v, ef_v, out_v):
    c = lax.axis_index("c")
    s = lax.axis_index("s")
    wid = s * _NC + c
    base = wid * _ET32

    pltpu.sync_copy(su_hbm, su_v)
    pltpu.sync_copy(sf_hbm, sf_v)
    pltpu.sync_copy(eu_hbm.at[pl.ds(base, _ET32)], eu_v)
    pltpu.sync_copy(ef_hbm.at[pl.ds(base, _ET32)], ef_v)

    def vec(i, _):
        eu16 = eu_v[pl.ds(16 * i, 16)]
        ef16 = ef_v[pl.ds(16 * i, 16)]
        a = plsc.load_gather(su_v, [eu16])
        b = plsc.load_gather(sf_v, [ef16])
        out_v[pl.ds(16 * i, 16)] = MAX_RATING / (1.0 + jnp.exp(-(a + b)))
        return _
    lax.fori_loop(0, _ET32 // 16, vec, None)
    pltpu.sync_copy(out_v, out_hbm.at[pl.ds(base, _ET32)])


def _sc_score(su, sf, eu_p, ef_p):
    f = pl.kernel(
        _sc_score_body,
        out_type=jax.ShapeDtypeStruct((E_PAD2,), _f32),
        mesh=_MESH,
        compiler_params=pltpu.CompilerParams(needs_layout_passes=False),
        scratch_types=[
            pltpu.VMEM((N_PAD,), _f32),
            pltpu.VMEM((N_PAD,), _f32),
            pltpu.VMEM((_ET32,), _i32),
            pltpu.VMEM((_ET32,), _i32),
            pltpu.VMEM((_ET32,), _f32),
        ],
    )
    return f(su, sf, eu_p, ef_p)


def kernel(u_feat, f_feat, edge_index, edge_w, edge_u, edge_f,
           Wu, bu, Wf, bf, W1, b1, W2, b2, g1, be1, g2, be2, Wr, br):
    row = edge_index[0]
    col = edge_index[1]

    ep = E_PAD - E
    row_p = jnp.concatenate([row, jnp.zeros((ep,), _i32)])
    col_p = jnp.concatenate([col, jnp.full((ep,), 10200, _i32)])
    ew_p = jnp.concatenate([edge_w, jnp.zeros((ep,), _f32)])
    row2 = row_p.reshape(E_PAD // 128, 128)
    col2 = col_p.reshape(E_PAD // 128, 128)
    ew2 = ew_p.reshape(E_PAD // 128, 128)

    ep2 = E_PAD2 - E
    eu_p = jnp.concatenate([edge_u, jnp.zeros((ep2,), _i32)])
    ef_p = jnp.concatenate([edge_f, jnp.full((ep2,), N_U, _i32)])

    u_pad = jnp.concatenate([u_feat, jnp.zeros((120, u_feat.shape[1]), _f32)])
    f_pad = jnp.concatenate([f_feat, jnp.zeros((120, f_feat.shape[1]), _f32)])
    xu = _mm_bias(u_pad, Wu, bu)
    xf = _mm_bias(f_pad, Wf, bf)
    x0 = jnp.concatenate([xu[:N_U], xf[:N_U],
                          jnp.zeros((N_PAD - N, HID), _f32)], axis=0)
    h1 = _chunk_mm(x0, W1)

    deg_parts = _sc_deg(col2, ew2)
    dinv2, norm2 = _sc_norm(deg_parts, row2, col2, ew2)
    dinv = dinv2[:N_PAD]

    normf = norm2.reshape(-1)

    def _msg(h4):
        hh = h4.reshape(NCHUNK, N_PAD, FC)
        out = []
        for cc in range(NCHUNK):
            msg = hh[cc][row_p] * normf[:, None]
            out.append(jnp.zeros((N_PAD, FC), _f32).at[col_p].add(msg))
        return jnp.stack(out)

    scat1 = _msg(h1)
    h2 = _post1(scat1.reshape(NCHUNK, N_PAD, FC), h1, dinv, b1, g1, be1, W2)

    scat2 = _msg(h2)
    wr2 = Wr.reshape(2, HID, 1)
    wr2 = jnp.concatenate([wr2[0], wr2[1]], axis=1)
    su, sf = _post2(scat2.reshape(NCHUNK, N_PAD, FC), h2, dinv,
                    b2, g2, be2, wr2, br)

    score_p = _sc_score(su.reshape(-1), sf.reshape(-1), eu_p, ef_p)
    return score_p[:E]


# speedup vs baseline: 4.4673x; 3.9921x over previous
import jax
import jax.numpy as jnp
from jax import lax
from jax.experimental import pallas as pl
from jax.experimental.pallas import tpu as pltpu
from jax.experimental.pallas import tpu_sc as plsc

N_U = 5000
N = 10000
E = 160000
HID = 512
FC = 128
NCHUNK = HID // FC
N_PAD = 10240
E_PAD = 163840
E_PAD2 = 160256
LRELU = 0.01
LN_EPS = 1e-5
MAX_RATING = 5.0

_f32 = jnp.float32
_i32 = jnp.int32


def _mm_bias_body(x_ref, w_ref, b_ref, o_ref):
    o_ref[...] = jnp.dot(x_ref[...], w_ref[...],
                         preferred_element_type=_f32) + b_ref[...]


def _mm_bias(x, w, b, blk=256):
    m = x.shape[0]
    return pl.pallas_call(
        _mm_bias_body,
        grid=(m // blk,),
        in_specs=[
            pl.BlockSpec((blk, x.shape[1]), lambda i: (i, 0)),
            pl.BlockSpec((w.shape[0], w.shape[1]), lambda i: (0, 0)),
            pl.BlockSpec((1, w.shape[1]), lambda i: (0, 0)),
        ],
        out_specs=pl.BlockSpec((blk, w.shape[1]), lambda i: (i, 0)),
        out_shape=jax.ShapeDtypeStruct((m, w.shape[1]), _f32),
    )(x, w, b.reshape(1, -1))


def _chunk_mm_body(x_ref, w_ref, o_ref):
    o_ref[0] = jnp.dot(x_ref[...], w_ref[...], preferred_element_type=_f32)


def _chunk_mm(x, w, blk=256):
    return pl.pallas_call(
        _chunk_mm_body,
        grid=(N_PAD // blk, NCHUNK),
        in_specs=[
            pl.BlockSpec((blk, HID), lambda m, c: (m, 0)),
            pl.BlockSpec((HID, FC), lambda m, c: (0, c)),
        ],
        out_specs=pl.BlockSpec((1, blk, FC), lambda m, c: (c, m, 0)),
        out_shape=jax.ShapeDtypeStruct((NCHUNK, N_PAD, FC), _f32),
    )(x, w)


def _gcn_epilogue(s_ref, h_ref, d_ref, b_ref, g_ref, be_ref):
    s = jnp.concatenate([s_ref[i] for i in range(NCHUNK)], axis=-1)
    h = jnp.concatenate([h_ref[i] for i in range(NCHUNK)], axis=-1)
    di = d_ref[0, 0]
    gcn = s + (di * di)[:, None] * h + b_ref[...]
    mu = jnp.mean(gcn, axis=-1, keepdims=True)
    var = jnp.mean((gcn - mu) ** 2, axis=-1, keepdims=True)
    xn = (gcn - mu) * lax.rsqrt(var + LN_EPS) * g_ref[...] + be_ref[...]
    return jnp.where(xn >= 0, xn, LRELU * xn)


def _post1_body(s_ref, h_ref, d_ref, b_ref, g_ref, be_ref, w_ref, o_ref):
    x = _gcn_epilogue(s_ref, h_ref, d_ref, b_ref, g_ref, be_ref)
    y = jnp.dot(x, w_ref[...], preferred_element_type=_f32)
    for c in range(NCHUNK):
        o_ref[c] = y[:, c * FC:(c + 1) * FC]


def _post1(scat, h, dinv, b, g, be, w_next, blk=256):
    return pl.pallas_call(
        _post1_body,
        grid=(N_PAD // blk,),
        in_specs=[
            pl.BlockSpec((NCHUNK, blk, FC), lambda m: (0, m, 0)),
            pl.BlockSpec((NCHUNK, blk, FC), lambda m: (0, m, 0)),
            pl.BlockSpec((1, 1, blk), lambda m: (m, 0, 0)),
            pl.BlockSpec((1, HID), lambda m: (0, 0)),
            pl.BlockSpec((1, HID), lambda m: (0, 0)),
            pl.BlockSpec((1, HID), lambda m: (0, 0)),
            pl.BlockSpec((HID, HID), lambda m: (0, 0)),
        ],
        out_specs=pl.BlockSpec((NCHUNK, blk, FC), lambda m: (0, m, 0)),
        out_shape=jax.ShapeDtypeStruct((NCHUNK, N_PAD, FC), _f32),
    )(scat, h, dinv.reshape(N_PAD // blk, 1, blk), b.reshape(1, -1),
      g.reshape(1, -1), be.reshape(1, -1), w_next)


def _post2_body(s_ref, h_ref, d_ref, b_ref, g_ref, be_ref, w_ref, br_ref,
                su_ref, sf_ref):
    x = _gcn_epilogue(s_ref, h_ref, d_ref, b_ref, g_ref, be_ref)
    y = jnp.dot(x, w_ref[...], preferred_element_type=_f32)
    su_ref[...] = y[:, 0:1] + br_ref[...]
    sf_ref[...] = y[:, 1:2]


def _post2(scat, h, dinv, b, g, be, wr2, br, blk=256):
    return pl.pallas_call(
        _post2_body,
        grid=(N_PAD // blk,),
        in_specs=[
            pl.BlockSpec((NCHUNK, blk, FC), lambda m: (0, m, 0)),
            pl.BlockSpec((NCHUNK, blk, FC), lambda m: (0, m, 0)),
            pl.BlockSpec((1, 1, blk), lambda m: (m, 0, 0)),
            pl.BlockSpec((1, HID), lambda m: (0, 0)),
            pl.BlockSpec((1, HID), lambda m: (0, 0)),
            pl.BlockSpec((1, HID), lambda m: (0, 0)),
            pl.BlockSpec((HID, 2), lambda m: (0, 0)),
            pl.BlockSpec((1, 1), lambda m: (0, 0)),
        ],
        out_specs=[
            pl.BlockSpec((blk, 1), lambda m: (m, 0)),
            pl.BlockSpec((blk, 1), lambda m: (m, 0)),
        ],
        out_shape=[
            jax.ShapeDtypeStruct((N_PAD, 1), _f32),
            jax.ShapeDtypeStruct((N_PAD, 1), _f32),
        ],
    )(scat, h, dinv.reshape(N_PAD // blk, 1, blk), b.reshape(1, -1),
      g.reshape(1, -1), be.reshape(1, -1), wr2, br.reshape(1, 1))


_MESH = plsc.VectorSubcoreMesh(core_axis_name="c", subcore_axis_name="s",
                               num_cores=2, num_subcores=16)
_NC, _NS = 2, 16
_NW = _NC * _NS

_EB32 = E_PAD // _NW // 128
_ROWS_T = N_PAD // _NS


def _newton_rsqrt(x):
    i = plsc.bitcast(x, _i32)
    i = jnp.full((16,), 0x5F3759DF, _i32) - (i >> 1)
    y = plsc.bitcast(i, _f32)
    for _ in range(3):
        y = y * (1.5 - 0.5 * x * y * y)
    return y


def _sc_deg_body(col3_hbm, ew_hbm, deg_hbm, col_v, ew_v, val_v, zv_v, acc_sh):
    c = lax.axis_index("c")
    s = lax.axis_index("s")
    wid = s * _NC + c

    def z16(m, _):
        zv_v[pl.ds(16 * m, 16)] = jnp.zeros((16,), _f32)
        return _
    lax.fori_loop(0, 8, z16, None)
    for p in range(_ROWS_T // 128):
        pltpu.sync_copy(zv_v, acc_sh.at[pl.ds(s * _ROWS_T + p * 128, 128)])
    plsc.subcore_barrier()

    pltpu.sync_copy(col3_hbm.at[pl.ds(wid * _EB32, _EB32)], col_v)
    pltpu.sync_copy(ew_hbm.at[pl.ds(wid * _EB32 * 128, _EB32 * 128)], ew_v)

    def blk(j, _):
        def cpv(m, _):
            val_v[pl.ds(16 * m, 16)] = ew_v[pl.ds(j * 128 + 16 * m, 16)]
            return _
        lax.fori_loop(0, 8, cpv, None)
        pltpu.sync_copy(val_v, acc_sh.at[col_v.at[j]], add=True)
        return _
    lax.fori_loop(0, _EB32, blk, None)
    plsc.subcore_barrier()

    for p in range(_ROWS_T // 128):
        pltpu.sync_copy(acc_sh.at[pl.ds(s * _ROWS_T + p * 128, 128)], zv_v)
        pltpu.sync_copy(
            zv_v,
            deg_hbm.at[pl.ds(c * N_PAD + s * _ROWS_T + p * 128, 128)])


def _sc_deg(col2, ew2):
    f = pl.kernel(
        _sc_deg_body,
        out_type=jax.ShapeDtypeStruct((2 * N_PAD,), _f32),
        mesh=_MESH,
        compiler_params=pltpu.CompilerParams(needs_layout_passes=False),
        scratch_types=[
            pltpu.VMEM((_EB32, 128), _i32),
            pltpu.VMEM((_EB32 * 128,), _f32),
            pltpu.VMEM((128,), _f32),
            pltpu.VMEM((128,), _f32),
            pltpu.VMEM_SHARED((N_PAD,), _f32),
        ],
    )
    return f(col2, ew2.reshape(-1))


def _sc_norm_body(deg_hbm, row_hbm, col_hbm, ew_hbm, dinv_hbm, norm_hbm,
                  v0_v, v1_v, dmine_v, dfull_v, row_v, col_v, ew_v, norm_v):
    c = lax.axis_index("c")
    s = lax.axis_index("s")
    wid = s * _NC + c

    pltpu.sync_copy(deg_hbm.at[pl.ds(s * _ROWS_T, _ROWS_T)], v0_v)
    pltpu.sync_copy(deg_hbm.at[pl.ds(N_PAD + s * _ROWS_T, _ROWS_T)], v1_v)

    def dv(v, _):
        d = v0_v[pl.ds(v * 16, 16)] + v1_v[pl.ds(v * 16, 16)] + 1.0
        dmine_v[pl.ds(v * 16, 16)] = _newton_rsqrt(d)
        return _
    lax.fori_loop(0, _ROWS_T // 16, dv, None)

    pltpu.sync_copy(dmine_v,
                    dinv_hbm.at[pl.ds(c * N_PAD + s * _ROWS_T, _ROWS_T)])
    plsc.subcore_barrier()
    pltpu.sync_copy(dinv_hbm.at[pl.ds(c * N_PAD, N_PAD)], dfull_v)

    pltpu.sync_copy(row_hbm.at[pl.ds(wid * _EB32, _EB32)], row_v)
    pltpu.sync_copy(col_hbm.at[pl.ds(wid * _EB32, _EB32)], col_v)
    pltpu.sync_copy(ew_hbm.at[pl.ds(wid * _EB32, _EB32)], ew_v)

    def blk(j, _):
        for m in range(8):
            r16 = row_v[j, pl.ds(16 * m, 16)]
            c16 = col_v[j, pl.ds(16 * m, 16)]
            w16 = ew_v[j, pl.ds(16 * m, 16)]
            dr = plsc.load_gather(dfull_v, [r16])
            dc = plsc.load_gather(dfull_v, [c16])
            norm_v[j, pl.ds(16 * m, 16)] = dr * w16 * dc
        return _
    lax.fori_loop(0, _EB32, blk, None)
    pltpu.sync_copy(norm_v, norm_hbm.at[pl.ds(wid * _EB32, _EB32)])


def _sc_norm(deg_parts, row2, col2, ew2):
    f = pl.kernel(
        _sc_norm_body,
        out_type=[
            jax.ShapeDtypeStruct((2 * N_PAD,), _f32),
            jax.ShapeDtypeStruct((E_PAD // 128, 128), _f32),
        ],
        mesh=_MESH,
        compiler_params=pltpu.CompilerParams(needs_layout_passes=False),
        scratch_types=[
            pltpu.VMEM((_ROWS_T,), _f32),
            pltpu.VMEM((_ROWS_T,), _f32),
            pltpu.VMEM((_ROWS_T,), _f32),
            pltpu.VMEM((N_PAD,), _f32),
            pltpu.VMEM((_EB32, 128), _i32),
            pltpu.VMEM((_EB32, 128), _i32),
            pltpu.VMEM((_EB32, 128), _f32),
            pltpu.VMEM((_EB32, 128), _f32),
        ],
    )
    return f(deg_parts, row2, col2, ew2)


_EB16 = E_PAD // _NS // 128


def _sc_msg_body(h_hbm, norm_hbm, row_hbm, col_hbm, scat_hbm,
                 row_v, col_v, norm_v, rows_v, sem, acc_sh):
    c = lax.axis_index("c")
    s = lax.axis_index("s")

    pltpu.sync_copy(row_hbm.at[pl.ds(s * _EB16, _EB16)], row_v)
    pltpu.sync_copy(col_hbm.at[pl.ds(s * _EB16, _EB16)], col_v)
    pltpu.sync_copy(norm_hbm.at[pl.ds(s * _EB16 * 128, _EB16 * 128)], norm_v)

    z16 = jnp.zeros((16,), _f32)

    for k in range(NCHUNK // _NC):
        chunk = c * (NCHUNK // _NC) + k
        off = (c * (NCHUNK // _NC) * N_PAD) if k == 0 else N_PAD

        def mkidx(j, _):
            for m in range(8):
                row_v[j, pl.ds(16 * m, 16)] = row_v[j, pl.ds(16 * m, 16)] + off
            return _
        lax.fori_loop(0, _EB16, mkidx, None)

        def zrow(j, _):
            for m in range(FC // 16):
                rows_v[j, 0, pl.ds(16 * m, 16)] = z16
            return _
        lax.fori_loop(0, 128, zrow, None)
        for p in range(_ROWS_T // 128):
            pltpu.sync_copy(rows_v, acc_sh.at[pl.ds(s * _ROWS_T + p * 128, 128)])
        plsc.subcore_barrier()

        def blk(j, _):
            pltpu.async_copy(h_hbm.at[row_v.at[j]], rows_v, sem).wait()

            def scale(e, _):
                nb = plsc.load_gather(norm_v,
                                      [jnp.full((16,), j * 128 + e, _i32)])
                for f in range(FC // 16):
                    rows_v[e, 0, pl.ds(16 * f, 16)] = (
                        rows_v[e, 0, pl.ds(16 * f, 16)] * nb)
                return _
            lax.fori_loop(0, 128, scale, None)
            pltpu.sync_copy(rows_v, acc_sh.at[col_v.at[j]], add=True)
            return _
        lax.fori_loop(0, _EB16, blk, None)
        plsc.subcore_barrier()

        coff = chunk * N_PAD
        for p in range(_ROWS_T // 128):
            pltpu.sync_copy(acc_sh.at[pl.ds(s * _ROWS_T + p * 128, 128)],
                            rows_v)
            pltpu.sync_copy(
                rows_v,
                scat_hbm.at[pl.ds(coff + s * _ROWS_T + p * 128, 128)])


def _sc_msg(h3, norm2, row2, col2):
    f = pl.kernel(
        _sc_msg_body,
        out_type=jax.ShapeDtypeStruct((NCHUNK * N_PAD, 1, FC), _f32),
        mesh=_MESH,
        compiler_params=pltpu.CompilerParams(needs_layout_passes=False),
        scratch_types=[
            pltpu.VMEM((_EB16, 128), _i32),
            pltpu.VMEM((_EB16, 128), _i32),
            pltpu.VMEM((_EB16 * 128,), _f32),
            pltpu.VMEM((128, 1, FC), _f32),
            pltpu.SemaphoreType.DMA,
            pltpu.VMEM_SHARED((N_PAD, 1, FC), _f32),
        ],
    )
    return f(h3, norm2.reshape(-1), row2, col2)


_ET32 = E_PAD2 // _NW


def _sc_score_body(su_hbm, sf_hbm, eu_hbm, ef_hbm, out_hbm,
                   su_v, sf_v, eu_v, ef_v, out_v):
    c = lax.axis_index("c")
    s = lax.axis_index("s")
    wid = s * _NC + c
    base = wid * _ET32

    pltpu.sync_copy(su_hbm, su_v)
    pltpu.sync_copy(sf_hbm, sf_v)
    pltpu.sync_copy(eu_hbm.at[pl.ds(base, _ET32)], eu_v)
    pltpu.sync_copy(ef_hbm.at[pl.ds(base, _ET32)], ef_v)

    def vec(i, _):
        eu16 = eu_v[pl.ds(16 * i, 16)]
        ef16 = ef_v[pl.ds(16 * i, 16)]
        a = plsc.load_gather(su_v, [eu16])
        b = plsc.load_gather(sf_v, [ef16])
        out_v[pl.ds(16 * i, 16)] = MAX_RATING / (1.0 + jnp.exp(-(a + b)))
        return _
    lax.fori_loop(0, _ET32 // 16, vec, None)
    pltpu.sync_copy(out_v, out_hbm.at[pl.ds(base, _ET32)])


def _sc_score(su, sf, eu_p, ef_p):
    f = pl.kernel(
        _sc_score_body,
        out_type=jax.ShapeDtypeStruct((E_PAD2,), _f32),
        mesh=_MESH,
        compiler_params=pltpu.CompilerParams(needs_layout_passes=False),
        scratch_types=[
            pltpu.VMEM((N_PAD,), _f32),
            pltpu.VMEM((N_PAD,), _f32),
            pltpu.VMEM((_ET32,), _i32),
            pltpu.VMEM((_ET32,), _i32),
            pltpu.VMEM((_ET32,), _f32),
        ],
    )
    return f(su, sf, eu_p, ef_p)


def kernel(u_feat, f_feat, edge_index, edge_w, edge_u, edge_f,
           Wu, bu, Wf, bf, W1, b1, W2, b2, g1, be1, g2, be2, Wr, br):
    row = edge_index[0]
    col = edge_index[1]

    ep = E_PAD - E
    row_p = jnp.concatenate([row, jnp.zeros((ep,), _i32)])
    col_p = jnp.concatenate([col, jnp.full((ep,), 10200, _i32)])
    ew_p = jnp.concatenate([edge_w, jnp.zeros((ep,), _f32)])
    row2 = row_p.reshape(E_PAD // 128, 128)
    col2 = col_p.reshape(E_PAD // 128, 128)
    ew2 = ew_p.reshape(E_PAD // 128, 128)

    ep2 = E_PAD2 - E
    eu_p = jnp.concatenate([edge_u, jnp.zeros((ep2,), _i32)])
    ef_p = jnp.concatenate([edge_f, jnp.full((ep2,), N_U, _i32)])

    u_pad = jnp.concatenate([u_feat, jnp.zeros((120, u_feat.shape[1]), _f32)])
    f_pad = jnp.concatenate([f_feat, jnp.zeros((120, f_feat.shape[1]), _f32)])
    xu = _mm_bias(u_pad, Wu, bu)
    xf = _mm_bias(f_pad, Wf, bf)
    x0 = jnp.concatenate([xu[:N_U], xf[:N_U],
                          jnp.zeros((N_PAD - N, HID), _f32)], axis=0)
    h1 = _chunk_mm(x0, W1)

    deg_parts = _sc_deg(col2, ew2)
    dinv2, norm2 = _sc_norm(deg_parts, row2, col2, ew2)
    dinv = dinv2[:N_PAD]

    scat1 = _sc_msg(h1.reshape(NCHUNK * N_PAD, 1, FC), norm2, row2, col2)
    h2 = _post1(scat1.reshape(NCHUNK, N_PAD, FC), h1, dinv, b1, g1, be1, W2)

    scat2 = _sc_msg(h2.reshape(NCHUNK * N_PAD, 1, FC), norm2, row2, col2)
    wr2 = Wr.reshape(2, HID, 1)
    wr2 = jnp.concatenate([wr2[0], wr2[1]], axis=1)
    su, sf = _post2(scat2.reshape(NCHUNK, N_PAD, FC), h2, dinv,
                    b2, g2, be2, wr2, br)

    score_p = _sc_score(su.reshape(-1), sf.reshape(-1), eu_p, ef_p)
    return score_p[:E]


# speedup vs baseline: 5.6878x; 1.2732x over previous
import jax
import jax.numpy as jnp
from jax import lax
from jax.experimental import pallas as pl
from jax.experimental.pallas import tpu as pltpu
from jax.experimental.pallas import tpu_sc as plsc

N_U = 5000
N = 10000
E = 160000
HID = 512
FC = 128
NCHUNK = HID // FC
N_PAD = 10240
E_PAD = 163840
E_PAD2 = 160256
LRELU = 0.01
LN_EPS = 1e-5
MAX_RATING = 5.0

_f32 = jnp.float32
_i32 = jnp.int32


def _mm_bias_body(x_ref, w_ref, b_ref, o_ref):
    o_ref[...] = jnp.dot(x_ref[...], w_ref[...],
                         preferred_element_type=_f32) + b_ref[...]


def _mm_bias(x, w, b, blk=256):
    m = x.shape[0]
    return pl.pallas_call(
        _mm_bias_body,
        grid=(m // blk,),
        in_specs=[
            pl.BlockSpec((blk, x.shape[1]), lambda i: (i, 0)),
            pl.BlockSpec((w.shape[0], w.shape[1]), lambda i: (0, 0)),
            pl.BlockSpec((1, w.shape[1]), lambda i: (0, 0)),
        ],
        out_specs=pl.BlockSpec((blk, w.shape[1]), lambda i: (i, 0)),
        out_shape=jax.ShapeDtypeStruct((m, w.shape[1]), _f32),
    )(x, w, b.reshape(1, -1))


def _chunk_mm_body(x_ref, w_ref, o_ref):
    o_ref[0] = jnp.dot(x_ref[...], w_ref[...], preferred_element_type=_f32)


def _chunk_mm(x, w, blk=256):
    return pl.pallas_call(
        _chunk_mm_body,
        grid=(N_PAD // blk, NCHUNK),
        in_specs=[
            pl.BlockSpec((blk, HID), lambda m, c: (m, 0)),
            pl.BlockSpec((HID, FC), lambda m, c: (0, c)),
        ],
        out_specs=pl.BlockSpec((1, blk, FC), lambda m, c: (c, m, 0)),
        out_shape=jax.ShapeDtypeStruct((NCHUNK, N_PAD, FC), _f32),
    )(x, w)


def _gcn_epilogue(s_ref, h_ref, d_ref, b_ref, g_ref, be_ref):
    s = jnp.concatenate([s_ref[i] for i in range(NCHUNK)], axis=-1)
    h = jnp.concatenate([h_ref[i] for i in range(NCHUNK)], axis=-1)
    di = d_ref[0, 0]
    gcn = s + (di * di)[:, None] * h + b_ref[...]
    mu = jnp.mean(gcn, axis=-1, keepdims=True)
    var = jnp.mean((gcn - mu) ** 2, axis=-1, keepdims=True)
    xn = (gcn - mu) * lax.rsqrt(var + LN_EPS) * g_ref[...] + be_ref[...]
    return jnp.where(xn >= 0, xn, LRELU * xn)


def _post1_body(s_ref, h_ref, d_ref, b_ref, g_ref, be_ref, w_ref, o_ref):
    x = _gcn_epilogue(s_ref, h_ref, d_ref, b_ref, g_ref, be_ref)
    y = jnp.dot(x, w_ref[...], preferred_element_type=_f32)
    for c in range(NCHUNK):
        o_ref[c] = y[:, c * FC:(c + 1) * FC]


def _post1(scat, h, dinv, b, g, be, w_next, blk=256):
    return pl.pallas_call(
        _post1_body,
        grid=(N_PAD // blk,),
        in_specs=[
            pl.BlockSpec((NCHUNK, blk, FC), lambda m: (0, m, 0)),
            pl.BlockSpec((NCHUNK, blk, FC), lambda m: (0, m, 0)),
            pl.BlockSpec((1, 1, blk), lambda m: (m, 0, 0)),
            pl.BlockSpec((1, HID), lambda m: (0, 0)),
            pl.BlockSpec((1, HID), lambda m: (0, 0)),
            pl.BlockSpec((1, HID), lambda m: (0, 0)),
            pl.BlockSpec((HID, HID), lambda m: (0, 0)),
        ],
        out_specs=pl.BlockSpec((NCHUNK, blk, FC), lambda m: (0, m, 0)),
        out_shape=jax.ShapeDtypeStruct((NCHUNK, N_PAD, FC), _f32),
    )(scat, h, dinv.reshape(N_PAD // blk, 1, blk), b.reshape(1, -1),
      g.reshape(1, -1), be.reshape(1, -1), w_next)


def _post2_body(s_ref, h_ref, d_ref, b_ref, g_ref, be_ref, w_ref, br_ref,
                su_ref, sf_ref):
    x = _gcn_epilogue(s_ref, h_ref, d_ref, b_ref, g_ref, be_ref)
    y = jnp.dot(x, w_ref[...], preferred_element_type=_f32)
    su_ref[...] = y[:, 0:1] + br_ref[...]
    sf_ref[...] = y[:, 1:2]


def _post2(scat, h, dinv, b, g, be, wr2, br, blk=256):
    return pl.pallas_call(
        _post2_body,
        grid=(N_PAD // blk,),
        in_specs=[
            pl.BlockSpec((NCHUNK, blk, FC), lambda m: (0, m, 0)),
            pl.BlockSpec((NCHUNK, blk, FC), lambda m: (0, m, 0)),
            pl.BlockSpec((1, 1, blk), lambda m: (m, 0, 0)),
            pl.BlockSpec((1, HID), lambda m: (0, 0)),
            pl.BlockSpec((1, HID), lambda m: (0, 0)),
            pl.BlockSpec((1, HID), lambda m: (0, 0)),
            pl.BlockSpec((HID, 2), lambda m: (0, 0)),
            pl.BlockSpec((1, 1), lambda m: (0, 0)),
        ],
        out_specs=[
            pl.BlockSpec((blk, 1), lambda m: (m, 0)),
            pl.BlockSpec((blk, 1), lambda m: (m, 0)),
        ],
        out_shape=[
            jax.ShapeDtypeStruct((N_PAD, 1), _f32),
            jax.ShapeDtypeStruct((N_PAD, 1), _f32),
        ],
    )(scat, h, dinv.reshape(N_PAD // blk, 1, blk), b.reshape(1, -1),
      g.reshape(1, -1), be.reshape(1, -1), wr2, br.reshape(1, 1))


_MESH = plsc.VectorSubcoreMesh(core_axis_name="c", subcore_axis_name="s",
                               num_cores=2, num_subcores=16)
_NC, _NS = 2, 16
_NW = _NC * _NS

_EB32 = E_PAD // _NW // 128
_ROWS_T = N_PAD // _NS


def _newton_rsqrt(x):
    i = plsc.bitcast(x, _i32)
    i = jnp.full((16,), 0x5F3759DF, _i32) - (i >> 1)
    y = plsc.bitcast(i, _f32)
    for _ in range(3):
        y = y * (1.5 - 0.5 * x * y * y)
    return y


def _sc_deg_body(col3_hbm, ew_hbm, deg_hbm, col_v, ew_v, val_v, zv_v, acc_sh):
    c = lax.axis_index("c")
    s = lax.axis_index("s")
    wid = s * _NC + c

    def z16(m, _):
        zv_v[pl.ds(16 * m, 16)] = jnp.zeros((16,), _f32)
        return _
    lax.fori_loop(0, 8, z16, None)
    for p in range(_ROWS_T // 128):
        pltpu.sync_copy(zv_v, acc_sh.at[pl.ds(s * _ROWS_T + p * 128, 128)])
    plsc.subcore_barrier()

    pltpu.sync_copy(col3_hbm.at[pl.ds(wid * _EB32, _EB32)], col_v)
    pltpu.sync_copy(ew_hbm.at[pl.ds(wid * _EB32 * 128, _EB32 * 128)], ew_v)

    def blk(j, _):
        def cpv(m, _):
            val_v[pl.ds(16 * m, 16)] = ew_v[pl.ds(j * 128 + 16 * m, 16)]
            return _
        lax.fori_loop(0, 8, cpv, None)
        pltpu.sync_copy(val_v, acc_sh.at[col_v.at[j]], add=True)
        return _
    lax.fori_loop(0, _EB32, blk, None)
    plsc.subcore_barrier()

    for p in range(_ROWS_T // 128):
        pltpu.sync_copy(acc_sh.at[pl.ds(s * _ROWS_T + p * 128, 128)], zv_v)
        pltpu.sync_copy(
            zv_v,
            deg_hbm.at[pl.ds(c * N_PAD + s * _ROWS_T + p * 128, 128)])


def _sc_deg(col2, ew2):
    f = pl.kernel(
        _sc_deg_body,
        out_type=jax.ShapeDtypeStruct((2 * N_PAD,), _f32),
        mesh=_MESH,
        compiler_params=pltpu.CompilerParams(needs_layout_passes=False),
        scratch_types=[
            pltpu.VMEM((_EB32, 128), _i32),
            pltpu.VMEM((_EB32 * 128,), _f32),
            pltpu.VMEM((128,), _f32),
            pltpu.VMEM((128,), _f32),
            pltpu.VMEM_SHARED((N_PAD,), _f32),
        ],
    )
    return f(col2, ew2.reshape(-1))


def _sc_norm_body(deg_hbm, row_hbm, col_hbm, ew_hbm, dinv_hbm, norm_hbm,
                  v0_v, v1_v, dmine_v, dfull_v, row_v, col_v, ew_v, norm_v):
    c = lax.axis_index("c")
    s = lax.axis_index("s")
    wid = s * _NC + c

    pltpu.sync_copy(deg_hbm.at[pl.ds(s * _ROWS_T, _ROWS_T)], v0_v)
    pltpu.sync_copy(deg_hbm.at[pl.ds(N_PAD + s * _ROWS_T, _ROWS_T)], v1_v)

    def dv(v, _):
        d = v0_v[pl.ds(v * 16, 16)] + v1_v[pl.ds(v * 16, 16)] + 1.0
        dmine_v[pl.ds(v * 16, 16)] = _newton_rsqrt(d)
        return _
    lax.fori_loop(0, _ROWS_T // 16, dv, None)

    pltpu.sync_copy(dmine_v,
                    dinv_hbm.at[pl.ds(c * N_PAD + s * _ROWS_T, _ROWS_T)])
    plsc.subcore_barrier()
    pltpu.sync_copy(dinv_hbm.at[pl.ds(c * N_PAD, N_PAD)], dfull_v)

    pltpu.sync_copy(row_hbm.at[pl.ds(wid * _EB32, _EB32)], row_v)
    pltpu.sync_copy(col_hbm.at[pl.ds(wid * _EB32, _EB32)], col_v)
    pltpu.sync_copy(ew_hbm.at[pl.ds(wid * _EB32, _EB32)], ew_v)

    def blk(j, _):
        for m in range(8):
            r16 = row_v[j, pl.ds(16 * m, 16)]
            c16 = col_v[j, pl.ds(16 * m, 16)]
            w16 = ew_v[j, pl.ds(16 * m, 16)]
            dr = plsc.load_gather(dfull_v, [r16])
            dc = plsc.load_gather(dfull_v, [c16])
            norm_v[j, pl.ds(16 * m, 16)] = dr * w16 * dc
        return _
    lax.fori_loop(0, _EB32, blk, None)
    pltpu.sync_copy(norm_v, norm_hbm.at[pl.ds(wid * _EB32, _EB32)])


def _sc_norm(deg_parts, row2, col2, ew2):
    f = pl.kernel(
        _sc_norm_body,
        out_type=[
            jax.ShapeDtypeStruct((2 * N_PAD,), _f32),
            jax.ShapeDtypeStruct((E_PAD // 128, 128), _f32),
        ],
        mesh=_MESH,
        compiler_params=pltpu.CompilerParams(needs_layout_passes=False),
        scratch_types=[
            pltpu.VMEM((_ROWS_T,), _f32),
            pltpu.VMEM((_ROWS_T,), _f32),
            pltpu.VMEM((_ROWS_T,), _f32),
            pltpu.VMEM((N_PAD,), _f32),
            pltpu.VMEM((_EB32, 128), _i32),
            pltpu.VMEM((_EB32, 128), _i32),
            pltpu.VMEM((_EB32, 128), _f32),
            pltpu.VMEM((_EB32, 128), _f32),
        ],
    )
    return f(deg_parts, row2, col2, ew2)


_EB16 = E_PAD // _NS // 128


def _sc_msg_body(h_hbm, norm_hbm, row_hbm, col_hbm, scat_hbm,
                 row_v, col0_v, col1_v, nrm0_v, nrm1_v, rows0_v, rows1_v,
                 sem0, sem1, semc0, semc1, semn0, semn1, acc_sh):
    c = lax.axis_index("c")
    s = lax.axis_index("s")

    pltpu.sync_copy(row_hbm.at[pl.ds(s * _EB16, _EB16)], row_v)

    z16 = jnp.zeros((16,), _f32)

    for k in range(NCHUNK // _NC):
        chunk = c * (NCHUNK // _NC) + k
        off = (c * (NCHUNK // _NC) * N_PAD) if k == 0 else N_PAD

        def mkidx(j, _):
            for m in range(8):
                row_v[j, pl.ds(16 * m, 16)] = row_v[j, pl.ds(16 * m, 16)] + off
            return _
        lax.fori_loop(0, _EB16, mkidx, None)

        def zrow(j, _):
            for m in range(FC // 16):
                rows0_v[j, 0, pl.ds(16 * m, 16)] = z16
            return _
        lax.fori_loop(0, 128, zrow, None)
        for p in range(_ROWS_T // 128):
            pltpu.sync_copy(rows0_v,
                            acc_sh.at[pl.ds(s * _ROWS_T + p * 128, 128)])
        plsc.subcore_barrier()

        def prefetch(j, buf, sem, colb, semc, nrmb, semn):
            pltpu.async_copy(h_hbm.at[row_v.at[j]], buf, sem)
            pltpu.async_copy(col_hbm.at[s * _EB16 + j], colb, semc)
            pltpu.async_copy(
                norm_hbm.at[pl.ds((s * _EB16 + j) * 128, 128)], nrmb, semn)

        prefetch(0, rows0_v, sem0, col0_v, semc0, nrm0_v, semn0)

        def half(j, buf, sem, colb, semc, nrmb, semn,
                 obuf, osem, ocolb, osemc, onrmb, osemn):
            pltpu.make_async_copy(h_hbm.at[row_v.at[0]], buf, sem).wait()
            pltpu.make_async_copy(col_hbm.at[0], colb, semc).wait()
            pltpu.make_async_copy(
                norm_hbm.at[pl.ds(0, 128)], nrmb, semn).wait()

            @pl.when(j + 1 < _EB16)
            def _():
                prefetch(j + 1, obuf, osem, ocolb, osemc, onrmb, osemn)

            def scale(e, _):
                nb = plsc.load_gather(nrmb, [jnp.full((16,), e, _i32)])
                for f in range(FC // 16):
                    buf[e, 0, pl.ds(16 * f, 16)] = (
                        buf[e, 0, pl.ds(16 * f, 16)] * nb)
                return _
            lax.fori_loop(0, 128, scale, None)
            pltpu.sync_copy(buf, acc_sh.at[colb], add=True)

        a0 = (rows0_v, sem0, col0_v, semc0, nrm0_v, semn0)
        a1 = (rows1_v, sem1, col1_v, semc1, nrm1_v, semn1)

        def blk(jj, _):
            half(2 * jj, *a0, *a1)
            half(2 * jj + 1, *a1, *a0)
            return _
        lax.fori_loop(0, _EB16 // 2, blk, None)
        plsc.subcore_barrier()

        coff = chunk * N_PAD
        for p in range(_ROWS_T // 128):
            pltpu.sync_copy(acc_sh.at[pl.ds(s * _ROWS_T + p * 128, 128)],
                            rows0_v)
            pltpu.sync_copy(
                rows0_v,
                scat_hbm.at[pl.ds(coff + s * _ROWS_T + p * 128, 128)])


def _sc_msg(h3, norm2, row2, col2):
    f = pl.kernel(
        _sc_msg_body,
        out_type=jax.ShapeDtypeStruct((NCHUNK * N_PAD, 1, FC), _f32),
        mesh=_MESH,
        compiler_params=pltpu.CompilerParams(needs_layout_passes=False),
        scratch_types=[
            pltpu.VMEM((_EB16, 128), _i32),
            pltpu.VMEM((128,), _i32),
            pltpu.VMEM((128,), _i32),
            pltpu.VMEM((128,), _f32),
            pltpu.VMEM((128,), _f32),
            pltpu.VMEM((128, 1, FC), _f32),
            pltpu.VMEM((128, 1, FC), _f32),
            pltpu.SemaphoreType.DMA,
            pltpu.SemaphoreType.DMA,
            pltpu.SemaphoreType.DMA,
            pltpu.SemaphoreType.DMA,
            pltpu.SemaphoreType.DMA,
            pltpu.SemaphoreType.DMA,
            pltpu.VMEM_SHARED((N_PAD, 1, FC), _f32),
        ],
    )
    return f(h3, norm2.reshape(-1), row2, col2)


_ET32 = E_PAD2 // _NW


def _sc_score_body(su_hbm, sf_hbm, eu_hbm, ef_hbm, out_hbm,
                   su_v, sf_v, eu_v, ef_v, out_v):
    c = lax.axis_index("c")
    s = lax.axis_index("s")
    wid = s * _NC + c
    base = wid * _ET32

    pltpu.sync_copy(su_hbm, su_v)
    pltpu.sync_copy(sf_hbm, sf_v)
    pltpu.sync_copy(eu_hbm.at[pl.ds(base, _ET32)], eu_v)
    pltpu.sync_copy(ef_hbm.at[pl.ds(base, _ET32)], ef_v)

    def vec(i, _):
        eu16 = eu_v[pl.ds(16 * i, 16)]
        ef16 = ef_v[pl.ds(16 * i, 16)]
        a = plsc.load_gather(su_v, [eu16])
        b = plsc.load_gather(sf_v, [ef16])
        out_v[pl.ds(16 * i, 16)] = MAX_RATING / (1.0 + jnp.exp(-(a + b)))
        return _
    lax.fori_loop(0, _ET32 // 16, vec, None)
    pltpu.sync_copy(out_v, out_hbm.at[pl.ds(base, _ET32)])


def _sc_score(su, sf, eu_p, ef_p):
    f = pl.kernel(
        _sc_score_body,
        out_type=jax.ShapeDtypeStruct((E_PAD2,), _f32),
        mesh=_MESH,
        compiler_params=pltpu.CompilerParams(needs_layout_passes=False),
        scratch_types=[
            pltpu.VMEM((N_PAD,), _f32),
            pltpu.VMEM((N_PAD,), _f32),
            pltpu.VMEM((_ET32,), _i32),
            pltpu.VMEM((_ET32,), _i32),
            pltpu.VMEM((_ET32,), _f32),
        ],
    )
    return f(su, sf, eu_p, ef_p)


def kernel(u_feat, f_feat, edge_index, edge_w, edge_u, edge_f,
           Wu, bu, Wf, bf, W1, b1, W2, b2, g1, be1, g2, be2, Wr, br):
    row = edge_index[0]
    col = edge_index[1]

    ep = E_PAD - E
    row_p = jnp.concatenate([row, jnp.zeros((ep,), _i32)])
    col_p = jnp.concatenate([col, jnp.full((ep,), 10200, _i32)])
    ew_p = jnp.concatenate([edge_w, jnp.zeros((ep,), _f32)])
    row2 = row_p.reshape(E_PAD // 128, 128)
    col2 = col_p.reshape(E_PAD // 128, 128)
    ew2 = ew_p.reshape(E_PAD // 128, 128)

    ep2 = E_PAD2 - E
    eu_p = jnp.concatenate([edge_u, jnp.zeros((ep2,), _i32)])
    ef_p = jnp.concatenate([edge_f, jnp.full((ep2,), N_U, _i32)])

    u_pad = jnp.concatenate([u_feat, jnp.zeros((120, u_feat.shape[1]), _f32)])
    f_pad = jnp.concatenate([f_feat, jnp.zeros((120, f_feat.shape[1]), _f32)])
    xu = _mm_bias(u_pad, Wu, bu)
    xf = _mm_bias(f_pad, Wf, bf)
    x0 = jnp.concatenate([xu[:N_U], xf[:N_U],
                          jnp.zeros((N_PAD - N, HID), _f32)], axis=0)
    h1 = _chunk_mm(x0, W1)

    deg_parts = _sc_deg(col2, ew2)
    dinv2, norm2 = _sc_norm(deg_parts, row2, col2, ew2)
    dinv = dinv2[:N_PAD]

    scat1 = _sc_msg(h1.reshape(NCHUNK * N_PAD, 1, FC), norm2, row2, col2)
    h2 = _post1(scat1.reshape(NCHUNK, N_PAD, FC), h1, dinv, b1, g1, be1, W2)

    scat2 = _sc_msg(h2.reshape(NCHUNK * N_PAD, 1, FC), norm2, row2, col2)
    wr2 = Wr.reshape(2, HID, 1)
    wr2 = jnp.concatenate([wr2[0], wr2[1]], axis=1)
    su, sf = _post2(scat2.reshape(NCHUNK, N_PAD, FC), h2, dinv,
                    b2, g2, be2, wr2, br)

    score_p = _sc_score(su.reshape(-1), sf.reshape(-1), eu_p, ef_p)
    return score_p[:E]
